# Initial kernel scaffold; baseline (speedup 1.0000x reference)
import jax
import jax.numpy as jnp
from jax.experimental import pallas as pl


def kernel(x, batch_idx, subset, adj, codebook, c_init, W_conv, b_conv, W_gt, b_gt, warm_up_rate):
    raise NotImplementedError("write your pallas kernel here")


# speedup vs baseline: 1.5642x; 1.5642x over previous
import functools

import jax
import jax.numpy as jnp
from jax.experimental import pallas as pl
from jax.experimental.pallas import tpu as pltpu

NUM_BRANCH = 4
NUM_D = 64
NUM_M = 1024


def _encode_body(x_ref, cbT_ref, cn_ref, enc_ref):
    x = x_ref[...]
    encs = []
    for i in range(NUM_BRANCH):
        xb = x[:, NUM_D * i:NUM_D * (i + 1)]
        cn = cn_ref[i, :][None, :]
        d = cn - 2.0 * jax.lax.dot_general(
            xb, cbT_ref[i], (((1,), (0,)), ((), ())),
            preferred_element_type=jnp.float32)
        m = jnp.min(d, axis=1, keepdims=True)
        iota = jax.lax.broadcasted_iota(jnp.int32, d.shape, 1)
        idx = jnp.min(jnp.where(d == m, iota, NUM_M), axis=1)
        encs.append(idx)
    lane8 = jax.lax.broadcasted_iota(jnp.int32, (x.shape[0], 8), 1)
    o = jnp.zeros((x.shape[0], 8), jnp.int32)
    for i in range(NUM_BRANCH):
        o = o + jnp.where(lane8 == i, encs[i][:, None], 0)
    enc_ref[...] = o


def _encode(x, codebook):
    Bn = x.shape[0]
    BR = 400
    cbT = codebook[:, :, :NUM_D].transpose(0, 2, 1)
    cn = jnp.sum(codebook[:, :, :NUM_D] ** 2, axis=2)
    return pl.pallas_call(
        _encode_body,
        grid=(Bn // BR,),
        in_specs=[
            pl.BlockSpec((BR, x.shape[1]), lambda r: (r, 0)),
            pl.BlockSpec(cbT.shape, lambda r: (0, 0, 0)),
            pl.BlockSpec(cn.shape, lambda r: (0, 0)),
        ],
        out_specs=pl.BlockSpec((BR, 8), lambda r: (r, 0)),
        out_shape=jax.ShapeDtypeStruct((Bn, 8), jnp.int32),
    )(x, cbT, cn)


def _matmul_body(x_ref, w_ref, b_ref, o_ref):
    o_ref[...] = jax.lax.dot_general(
        x_ref[...], w_ref[...], (((1,), (0,)), ((), ())),
        preferred_element_type=jnp.float32) + b_ref[...]


def _matmul_bias(x, w, b):
    n, k = x.shape
    m = w.shape[1]
    BR = 1000
    return pl.pallas_call(
        _matmul_body,
        grid=(n // BR,),
        in_specs=[
            pl.BlockSpec((BR, k), lambda r: (r, 0)),
            pl.BlockSpec((k, m), lambda r: (0, 0)),
            pl.BlockSpec((1, m), lambda r: (0, 0)),
        ],
        out_specs=pl.BlockSpec((BR, m), lambda r: (r, 0)),
        out_shape=jax.ShapeDtypeStruct((n, m), jnp.float32),
    )(x, w, b.reshape(1, m))


def _dotsum_body(a_ref, b_ref, o_ref):
    @pl.when(pl.program_id(0) == 0)
    def _init():
        o_ref[0, 0] = 0.0

    o_ref[0, 0] += jnp.sum(a_ref[...] * b_ref[...])


def _dotsum(a, b):
    n, c = a.shape
    BR = 1000
    out = pl.pallas_call(
        _dotsum_body,
        grid=(n // BR,),
        in_specs=[
            pl.BlockSpec((BR, c), lambda r: (r, 0)),
            pl.BlockSpec((BR, c), lambda r: (r, 0)),
        ],
        out_specs=pl.BlockSpec((1, 1), lambda r: (0, 0), memory_space=pltpu.SMEM),
        out_shape=jax.ShapeDtypeStruct((1, 1), jnp.float32),
    )(a, b)
    return out[0, 0]


import jax.lax as lax
from jax.experimental.pallas import tpu_sc as plsc

NC, NS, L = 2, 16, 16
NW = NC * NS
_N = 50000
_B = 10000
_F = 40000
_E = 300000

_MESH = plsc.VectorSubcoreMesh(core_axis_name="c", subcore_axis_name="s")
def _lane():
    return jax.lax.broadcasted_iota(jnp.int32, (L,), 0)


def _u32(v):
    return plsc.bitcast(v, jnp.uint32)


def _i32(v):
    return plsc.bitcast(v, jnp.int32)


def _kidx_body(bidx_hbm, fo_hbm, enc_hbm, cinit_hbm, asrc_hbm, adst_hbm,
               gidx_hbm, packed_hbm,
               bidx_v, pos_loc, es_v, ed_v, pk_v, stage_v,
               fo_v, p_v, eidx_v, cidx_v, eg_v, cg_v, gout_v,
               pos_sh, cin_sh, enc_sh):
    core = lax.axis_index("c")
    sub = lax.axis_index("s")
    wid = sub * NC + core

    eoff = jnp.minimum(wid * 9376, _E - 9376)
    pltpu.sync_copy(asrc_hbm.at[pl.ds(eoff, 9376)], es_v)
    pltpu.sync_copy(adst_hbm.at[pl.ds(eoff, 9376)], ed_v)

    def _pack(t, _):
        s_ = es_v[pl.ds(t * L, L)]
        d_ = ed_v[pl.ds(t * L, L)]
        pk_v[pl.ds(t * L, L)] = _i32((_u32(d_) << 16) | _u32(s_))
        return _
    lax.fori_loop(0, 9376 // L, _pack, 0)
    pltpu.sync_copy(pk_v, packed_hbm.at[pl.ds(eoff, 9376)])

    coff = jnp.minimum(sub * 12504, 200000 - 12504)
    pltpu.sync_copy(cinit_hbm.at[pl.ds(coff, 12504)], stage_v)
    pltpu.sync_copy(stage_v, cin_sh.at[pl.ds(coff, 12504)])
    eoff2 = jnp.minimum(sub * 5008, 80000 - 5008)
    pltpu.sync_copy(enc_hbm.at[pl.ds(eoff2, 5008)], stage_v.at[pl.ds(0, 5008)])
    pltpu.sync_copy(stage_v.at[pl.ds(0, 5008)], enc_sh.at[pl.ds(eoff2, 5008)])

    lo = sub * 3128
    pltpu.sync_copy(bidx_hbm, bidx_v)

    def _init(t, _):
        pos_loc[pl.ds(t * L, L)] = jnp.full((L,), -1, jnp.int32)
        return _
    lax.fori_loop(0, 3136 // L, _init, 0)

    def _scan(k, carry):
        v = bidx_v[pl.ds(k * L, L)]
        kvec = k * L + _lane()
        _cnt, lastocc = plsc.scan_count(v)
        owned = (v >= lo) & (v < lo + 3128)
        plsc.store_scatter(pos_loc, [v - lo], kvec, mask=lastocc & owned)
        return carry
    lax.fori_loop(0, _B // L, _scan, 0)
    pltpu.sync_copy(pos_loc.at[pl.ds(0, 3128)], pos_sh.at[pl.ds(lo, 3128)])

    plsc.subcore_barrier()

    j0 = jnp.minimum(wid * 1280, _F - 1280)
    pltpu.sync_copy(fo_hbm.at[pl.ds(j0, 1280)], fo_v)
    pltpu.sync_copy(pos_sh.at[fo_v], p_v)
    for i in range(NUM_BRANCH):
        def _mkidx(t, _):
            pv = p_v[pl.ds(t * L, L)]
            fv = fo_v[pl.ds(t * L, L)]
            eidx_v[pl.ds(t * L, L)] = jnp.maximum(pv, 0) * 8 + i
            cidx_v[pl.ds(t * L, L)] = fv + i * _N
            return _
        lax.fori_loop(0, 1280 // L, _mkidx, 0)
        pltpu.sync_copy(enc_sh.at[eidx_v], eg_v)
        pltpu.sync_copy(cin_sh.at[cidx_v], cg_v)

        def _comb(t, _):
            pv = p_v[pl.ds(t * L, L)]
            g = jnp.where(pv >= 0, eg_v[pl.ds(t * L, L)], cg_v[pl.ds(t * L, L)])
            gout_v[pl.ds(t * L, L)] = g + i * NUM_M
            return _
        lax.fori_loop(0, 1280 // L, _comb, 0)
        pltpu.sync_copy(gout_v, gidx_hbm.at[pl.ds(i * _F + j0, 1280)])


def _kidx(batch_idx, fo, enc_flat, cinit_flat, asrc, adst):
    f = pl.kernel(
        _kidx_body,
        out_type=[
            jax.ShapeDtypeStruct((NUM_BRANCH * _F,), jnp.int32),
            jax.ShapeDtypeStruct((_E + 32,), jnp.int32),
        ],
        mesh=_MESH,
        compiler_params=pltpu.CompilerParams(needs_layout_passes=False),
        scratch_types=[
            pltpu.VMEM((_B,), jnp.int32),
            pltpu.VMEM((3136,), jnp.int32),
            pltpu.VMEM((9376,), jnp.int32),
            pltpu.VMEM((9376,), jnp.int32),
            pltpu.VMEM((9376,), jnp.int32),
            pltpu.VMEM((12504,), jnp.int32),
            pltpu.VMEM((1280,), jnp.int32),
            pltpu.VMEM((1280,), jnp.int32),
            pltpu.VMEM((1280,), jnp.int32),
            pltpu.VMEM((1280,), jnp.int32),
            pltpu.VMEM((1280,), jnp.int32),
            pltpu.VMEM((1280,), jnp.int32),
            pltpu.VMEM((1280,), jnp.int32),
            pltpu.VMEM_SHARED((_N + 48,), jnp.int32),
            pltpu.VMEM_SHARED((200000,), jnp.int32),
            pltpu.VMEM_SHARED((80000,), jnp.int32),
        ],
    )
    return f(batch_idx, fo, enc_flat, cinit_flat, asrc, adst)


def _kh_body(hx_hbm, tcb_hbm, cbg_hbm, gidx_hbm,
             h_hbm, gp_hbm,
             rowbuf, rowbuf2, gbuf, gi_v):
    core = lax.axis_index("c")
    sub = lax.axis_index("s")
    wid = sub * NC + core

    for t in range(4):
        w = wid + t * NW
        @pl.when(w < 125)
        def _cp():
            pltpu.sync_copy(hx_hbm.at[pl.ds(w * 80, 80)], rowbuf.at[pl.ds(0, 80)])
            pltpu.sync_copy(rowbuf.at[pl.ds(0, 80)], h_hbm.at[pl.ds(w * 80, 80)])

    jbase = jnp.minimum(wid * 1280, _F - 1280)

    def _win(t, carry):
        j0 = jbase + t * 128
        for i in range(NUM_BRANCH):
            pltpu.sync_copy(gidx_hbm.at[pl.ds(i * _F + j0, 128)], gi_v)
            pltpu.sync_copy(tcb_hbm.at[gi_v], rowbuf if i == 0 else rowbuf2)
            if i != 0:
                def _acc(r, c2):
                    for q in range(16):
                        rowbuf[r, pl.ds(q * L, L)] = (
                            rowbuf[r, pl.ds(q * L, L)]
                            + rowbuf2[r, pl.ds(q * L, L)])
                    return c2
                lax.fori_loop(0, 128, _acc, 0)
            pltpu.sync_copy(cbg_hbm.at[gi_v], gbuf)
            pltpu.sync_copy(gbuf, gp_hbm.at[pl.ds(i * _F + j0, 128)])
        pltpu.sync_copy(rowbuf, h_hbm.at[pl.ds(_B + j0, 128)])
        return carry
    lax.fori_loop(0, 10, _win, 0)


def _kh(h_x, tcb, cbg, gidx_flat):
    f = pl.kernel(
        _kh_body,
        out_type=[
            jax.ShapeDtypeStruct((_N, 256), jnp.float32),
            jax.ShapeDtypeStruct((NUM_BRANCH * _F, 2 * NUM_D), jnp.float32),
        ],
        mesh=_MESH,
        compiler_params=pltpu.CompilerParams(needs_layout_passes=False),
        scratch_types=[
            pltpu.VMEM((128, 256), jnp.float32),
            pltpu.VMEM((128, 256), jnp.float32),
            pltpu.VMEM((128, 2 * NUM_D), jnp.float32),
            pltpu.VMEM((128,), jnp.int32),
        ],
    )
    return f(h_x, tcb, cbg, gidx_flat)


_CR = 4096
_NCHUNK = 13
_SEG = 4176


def _k3_body(packed_hbm, h_hbm, out_hbm,
             pk_v, sel_loc, seg_v, fine_sel, cnt_v, cnt_all,
             sbuf, dbuf, acc_f, rows_v,
             sel_sh, cnt_sh, sem_a):
    core = lax.axis_index("c")
    sub = lax.axis_index("s")

    eoff = sub * 18752
    nmine = jnp.minimum(18752, _E - eoff)
    pltpu.sync_copy(packed_hbm.at[pl.ds(eoff, 18752)], pk_v)

    def _chunk(c, carry0):
        @pl.when((c % 2) == core)
        def _run():
            base = c * _CR
            stripe = base + sub * 256

            def _sel(t, woff):
                u = _u32(pk_v[pl.ds(t * L, L)])
                d_ = _i32(u >> 16)
                valid = (t * L + _lane()) < nmine
                m = ((d_ >> 12) == c) & valid
                plsc.store_compressed(sel_loc.at[pl.ds(woff, L)],
                                      _i32(u), mask=m)
                pc = plsc.all_reduce_population_count(m)
                return jnp.minimum(woff + jnp.max(pc), 4096)
            woff = lax.fori_loop(0, 18752 // L, _sel, 0)
            cnt_v[...] = jnp.broadcast_to(woff, (L,)).astype(jnp.int32)
            pltpu.sync_copy(sel_loc, sel_sh.at[pl.ds(sub * _SEG, _SEG)])
            pltpu.sync_copy(cnt_v, cnt_sh.at[pl.ds(sub * L, L)])
            plsc.subcore_barrier()

            pltpu.sync_copy(cnt_sh, cnt_all)

            def _seg(s2, foff):
                pltpu.sync_copy(sel_sh.at[pl.ds(s2 * _SEG, _SEG)], seg_v)
                cnt = jnp.max(cnt_all[pl.ds(s2 * L, L)])
                nq = (cnt + L - 1) // L

                def _scan2(q, foff2):
                    v = seg_v[pl.ds(q * L, L)]
                    u = _u32(v)
                    d_ = _i32(u >> 16)
                    valid = (q * L + _lane()) < cnt
                    m = valid & (d_ >= stripe) & (d_ < stripe + 256)
                    plsc.store_compressed(fine_sel.at[pl.ds(foff2, L)],
                                          v, mask=m)
                    pc = plsc.all_reduce_population_count(m)
                    return jnp.minimum(foff2 + jnp.max(pc), 4096)
                return lax.fori_loop(0, nq, _scan2, foff)
            foff = lax.fori_loop(0, NS, _seg, 0)

            k0 = foff // L
            fillv = _i32(((_u32(stripe + 256 + (_lane() & 7))) << 16))
            for t in range(3):
                vi = k0 + t
                fill = (vi * L + _lane()) >= foff
                old = fine_sel[pl.ds(vi * L, L)]
                fine_sel[pl.ds(vi * L, L)] = jnp.where(fill, fillv, old)

            def _z(r, carry):
                acc_f[pl.ds(r * L, L)] = jnp.zeros((L,), jnp.float32)
                return carry
            lax.fori_loop(0, 264 * 256 // L, _z, 0)

            ng = (foff + 31) // 32
            lanev = _lane()

            def _grp(g, carry):
                for q2 in range(2):
                    v = fine_sel[pl.ds(g * 32 + q2 * L, L)]
                    u = _u32(v)
                    sbuf[pl.ds(q2 * L, L)] = _i32(u & 0xFFFF)
                    dbuf[pl.ds(q2 * L, L)] = _i32(u >> 16) - stripe
                pltpu.async_copy(h_hbm.at[sbuf], rows_v, sem_a).wait()

                def _acc_e(e, carry2):
                    q2b = e // L
                    lp = e % L
                    dvec = dbuf[pl.ds(q2b * L, L)]
                    dsp = jnp.take_along_axis(
                        dvec, jnp.broadcast_to(lp, (L,)).astype(jnp.int32), axis=0)
                    ibase = dsp * 256 + lanev
                    for w in range(16):
                        x = rows_v[e, pl.ds(w * L, L)]
                        plsc.addupdate_scatter(acc_f, [ibase + w * L], x)
                    return carry2
                lax.fori_loop(0, 32, _acc_e, 0)
                return carry
            lax.fori_loop(0, ng, _grp, 0)

            def _drblk(blk, carry):
                def _ld(r2, c2):
                    for q2 in range(256 // L):
                        rows_v[r2, pl.ds(q2 * L, L)] = acc_f[
                            pl.ds((blk * 32 + r2) * 256 + q2 * L, L)]
                    return c2
                lax.fori_loop(0, 32, _ld, 0)
                off = pl.multiple_of(base + sub * 256 + blk * 32, 32)
                pltpu.sync_copy(rows_v, out_hbm.at[pl.ds(off, 32)])
                return carry
            lax.fori_loop(0, 8, _drblk, 0)
            plsc.subcore_barrier()
        return carry0
    lax.fori_loop(0, _NCHUNK, _chunk, 0)


def _k3(packed, h_full):
    f = pl.kernel(
        _k3_body,
        out_type=[
            jax.ShapeDtypeStruct((_NCHUNK * _CR, 256), jnp.float32),
        ],
        mesh=_MESH,
        compiler_params=pltpu.CompilerParams(needs_layout_passes=False),
        scratch_types=[
            pltpu.VMEM((18752,), jnp.int32),
            pltpu.VMEM((_SEG,), jnp.int32),
            pltpu.VMEM((_SEG,), jnp.int32),
            pltpu.VMEM((_SEG,), jnp.int32),
            pltpu.VMEM((L,), jnp.int32),
            pltpu.VMEM((NS * L,), jnp.int32),
            pltpu.VMEM((32,), jnp.int32),
            pltpu.VMEM((32,), jnp.int32),
            pltpu.VMEM((264 * 256,), jnp.float32),
            pltpu.VMEM((32, 256), jnp.float32),
            pltpu.VMEM_SHARED((NS * _SEG,), jnp.int32),
            pltpu.VMEM_SHARED((NS * L,), jnp.int32),
            pltpu.SemaphoreType.DMA,
        ],
    )
    return f(packed, h_full)[0]


def _tcb_body(cb_ref, w_ref, o_ref):
    o_ref[...] = jax.lax.dot_general(
        cb_ref[0, :, :NUM_D], w_ref[...], (((1,), (0,)), ((), ())),
        preferred_element_type=jnp.float32)


def _tcb(codebook, W_conv):
    return pl.pallas_call(
        _tcb_body,
        grid=(NUM_BRANCH,),
        in_specs=[
            pl.BlockSpec((1, NUM_M, 2 * NUM_D), lambda r: (r, 0, 0)),
            pl.BlockSpec((NUM_D, 256), lambda r: (r, 0)),
        ],
        out_specs=pl.BlockSpec((NUM_M, 256), lambda r: (r, 0)),
        out_shape=jax.ShapeDtypeStruct((NUM_BRANCH * NUM_M, 256), jnp.float32),
    )(codebook, W_conv)


def _mm_out_body(x_ref, w_ref, bc_ref, bg_ref, o_ref):
    w = w_ref[...]
    b2 = jax.lax.dot_general(bc_ref[...], w, (((1,), (0,)), ((), ())),
                             preferred_element_type=jnp.float32) + bg_ref[...]
    o_ref[...] = jax.lax.dot_general(
        x_ref[...], w, (((1,), (0,)), ((), ())),
        preferred_element_type=jnp.float32) + b2


def _mm_out(out_s, W_gt, b_conv, b_gt):
    BR = 1000
    return pl.pallas_call(
        _mm_out_body,
        grid=(_B // BR,),
        in_specs=[
            pl.BlockSpec((BR, 256), lambda r: (r, 0)),
            pl.BlockSpec((256, 256), lambda r: (0, 0)),
            pl.BlockSpec((1, 256), lambda r: (0, 0)),
            pl.BlockSpec((1, 256), lambda r: (0, 0)),
        ],
        out_specs=pl.BlockSpec((BR, 256), lambda r: (r, 0)),
        out_shape=jax.ShapeDtypeStruct((_B, 256), jnp.float32),
    )(out_s, W_gt, b_conv.reshape(1, 256), b_gt.reshape(1, 256))


def _dotsum2_body(a_ref, g_ref, bc_ref, wur_ref, o_ref):
    gid = pl.program_id(0)

    @pl.when(gid == 0)
    def _init():
        o_ref[0, 0] = 0.0

    part = 0.0
    for i in range(NUM_BRANCH):
        part += jnp.sum((a_ref[:, i, :] + bc_ref[i, :][None, :])
                        * g_ref[i][:, NUM_D:])
    o_ref[0, 0] += part

    @pl.when(gid == pl.num_programs(0) - 1)
    def _fin():
        o_ref[0, 0] *= wur_ref[0]


def _dotsum2(out_s, g_parts, b_conv, warm_up_rate):
    BR = 1000
    nr = _F // BR
    a3 = out_s.reshape(out_s.shape[0], NUM_BRANCH, NUM_D)
    g3 = g_parts.reshape(NUM_BRANCH, _F, 2 * NUM_D)
    out = pl.pallas_call(
        _dotsum2_body,
        grid=(nr,),
        in_specs=[
            pl.BlockSpec((BR, NUM_BRANCH, NUM_D), lambda g: (_B // 1000 + g, 0, 0)),
            pl.BlockSpec((NUM_BRANCH, BR, 2 * NUM_D), lambda g: (0, g, 0)),
            pl.BlockSpec((NUM_BRANCH, NUM_D), lambda g: (0, 0)),
            pl.BlockSpec(memory_space=pltpu.SMEM),
        ],
        out_specs=pl.BlockSpec((1, 1), lambda g: (0, 0), memory_space=pltpu.SMEM),
        out_shape=jax.ShapeDtypeStruct((1, 1), jnp.float32),
    )(a3, g3, b_conv.reshape(NUM_BRANCH, NUM_D),
      warm_up_rate.reshape(1))
    return out[0, 0]


def kernel(x, batch_idx, subset, adj, codebook, c_init, W_conv, b_conv, W_gt, b_gt, warm_up_rate):
    Bn = x.shape[0]
    first_order_idx = subset[Bn:]

    enc = _encode(x, codebook)
    gidx_flat, packed = _kidx(batch_idx, first_order_idx, enc.reshape(-1),
                              c_init.reshape(-1), adj[0], adj[1])

    h_x = _matmul_bias(x, W_conv, jnp.zeros_like(b_conv))
    tcb = _tcb(codebook, W_conv)
    cbg = codebook.reshape(NUM_BRANCH * NUM_M, 2 * NUM_D)

    h_full, g_parts = _kh(h_x, tcb, cbg, gidx_flat)
    out_s = _k3(packed, h_full)

    x_out = _mm_out(out_s, W_gt, b_conv, b_gt)
    info_backward = _dotsum2(out_s, g_parts, b_conv, warm_up_rate)
    return (x_out, info_backward)
